# baseline (speedup 1.0000x reference)
import functools

import jax
import jax.numpy as jnp
from jax import lax
from jax.experimental import pallas as pl
from jax.experimental.pallas import tpu as pltpu
from jax.experimental.pallas import tpu_sc as plsc

N = 10000
E = 320000
D = 128
DE = 16
L = 3
G = 128
B = 64

NC, NS = 2, 16
NW = NC * NS
CH = 80
NBUF = 5
NPAD = 10240
RPS = NPAD // NS
DEP = 32
DH = D // 2
ESS = E // NS
NCH = ESS // CH
EPW = E // NW
NCHE = EPW // CH

_mesh = plsc.VectorSubcoreMesh(core_axis_name="c", subcore_axis_name="s")


@functools.partial(
    pl.kernel,
    mesh=_mesh,
    compiler_params=pltpu.CompilerParams(use_tc_tiling_on_sc=False),
    out_type=jax.ShapeDtypeStruct((NC, NPAD, DH), jnp.float32),
    scratch_types=(
        [pltpu.VMEM((NCH, CH), jnp.int32),
         pltpu.VMEM((NCH, CH), jnp.int32)]
        + [pltpu.VMEM((CH, DH), jnp.float32) for _ in range(NBUF)]
        + [pltpu.SemaphoreType.DMA for _ in range(2 * NBUF)]
        + [pltpu.VMEM_SHARED((NPAD, DH), jnp.float32)]
    ),
)
def _sc_row_scatter(h0_hbm, h1_hbm, srcr_hbm, dstr_hbm, zrow_hbm, out_hbm,
                    src_v, dst_v, b0, b1, b2, b3, b4, s0, s1, s2, s3, s4,
                    t0, t1, t2, t3, t4, acc_sh):
    bufs = (b0, b1, b2, b3, b4)
    sems = (s0, s1, s2, s3, s4)
    ssems = (t0, t1, t2, t3, t4)
    cid = lax.axis_index("c")
    sid = lax.axis_index("s")

    pltpu.sync_copy(srcr_hbm.at[sid], src_v)
    pltpu.sync_copy(dstr_hbm.at[sid], dst_v)
    pltpu.sync_copy(zrow_hbm, acc_sh.at[pl.ds(sid * RPS, RPS)])
    plsc.subcore_barrier()

    def gather(j, b):
        @pl.when(cid == 0)
        def _():
            pltpu.async_copy(h0_hbm.at[src_v.at[j]], bufs[b], sems[b])

        @pl.when(cid == 1)
        def _():
            pltpu.async_copy(h1_hbm.at[src_v.at[j]], bufs[b], sems[b])

    for b in range(NBUF):
        gather(b, b)

    def body(g, carry):
        for b in range(NBUF):
            j = g * NBUF + b
            pltpu.make_async_copy(h0_hbm.at[src_v.at[j]], bufs[b], sems[b]).wait()
            pltpu.async_copy(bufs[b], acc_sh.at[dst_v.at[j]], ssems[b], add=True)
            nxt = j + NBUF

            @pl.when(nxt < NCH)
            def _():
                pltpu.make_async_copy(bufs[b], acc_sh.at[dst_v.at[0]], ssems[b]).wait()
                gather(nxt, b)
        return carry

    lax.fori_loop(0, NCH // NBUF, body, 0)
    for b in range(NBUF):
        pltpu.make_async_copy(bufs[b], acc_sh.at[dst_v.at[0]], ssems[b]).wait()
    plsc.subcore_barrier()
    pltpu.sync_copy(acc_sh.at[pl.ds(sid * RPS, RPS)],
                    out_hbm.at[cid, pl.ds(sid * RPS, RPS)])


@functools.partial(
    pl.kernel,
    mesh=_mesh,
    compiler_params=pltpu.CompilerParams(use_tc_tiling_on_sc=False),
    out_type=jax.ShapeDtypeStruct((NC, NPAD, DEP), jnp.float32),
    scratch_types=(
        [pltpu.VMEM((NCHE, CH), jnp.int32)]
        + [pltpu.VMEM((CH, DEP), jnp.float32) for _ in range(NBUF)]
        + [pltpu.SemaphoreType.DMA for _ in range(NBUF)]
        + [pltpu.VMEM_SHARED((NPAD, DEP), jnp.float32)]
    ),
)
def _sc_ef_scatter(ef_hbm, dstr_hbm, zrow_hbm, out_hbm,
                   dst_v, b0, b1, b2, b3, b4, s0, s1, s2, s3, s4, acc_sh):
    bufs = (b0, b1, b2, b3, b4)
    sems = (s0, s1, s2, s3, s4)
    cid = lax.axis_index("c")
    sid = lax.axis_index("s")
    wid = sid * NC + cid

    pltpu.sync_copy(dstr_hbm.at[wid], dst_v)
    pltpu.sync_copy(zrow_hbm, acc_sh.at[pl.ds(sid * RPS, RPS)])
    plsc.subcore_barrier()

    for b in range(NBUF):
        pltpu.async_copy(ef_hbm.at[wid, b], bufs[b], sems[b])

    def body(g, carry):
        for b in range(NBUF):
            j = g * NBUF + b
            pltpu.make_async_copy(ef_hbm.at[wid, j], bufs[b], sems[b]).wait()
            pltpu.sync_copy(bufs[b], acc_sh.at[dst_v.at[j]], add=True)
            nxt = j + NBUF

            @pl.when(nxt < NCHE)
            def _():
                pltpu.async_copy(ef_hbm.at[wid, nxt], bufs[b], sems[b])
        return carry

    lax.fori_loop(0, NCHE // NBUF, body, 0)
    plsc.subcore_barrier()
    pltpu.sync_copy(acc_sh.at[pl.ds(sid * RPS, RPS)],
                    out_hbm.at[cid, pl.ds(sid * RPS, RPS)])


def _tc_split_body(x_ref, o_ref):
    o_ref[0] = x_ref[:, :DH]
    o_ref[1] = x_ref[:, DH:]


_tc_split = pl.pallas_call(
    _tc_split_body,
    out_shape=jax.ShapeDtypeStruct((2, N, DH), jnp.float32),
)


def _tc_layer_body(h_ref, sp_ref, efd_ref, wm_ref, bm_ref, wn_ref, bn_ref,
                   gam_ref, bet_ref, o_ref):
    h = h_ref[...]
    efd = (efd_ref[0] + efd_ref[1])[:N]
    ef = efd[:, :DE]
    deg = efd[:, DE:DE + 1]
    wm = wm_ref[...]
    aggr = jnp.dot(deg * h, wm[:D], preferred_element_type=jnp.float32, precision=lax.Precision.HIGHEST)
    aggr = aggr + jnp.dot(sp_ref[0, :N], wm[D:D + DH], preferred_element_type=jnp.float32, precision=lax.Precision.HIGHEST)
    aggr = aggr + jnp.dot(sp_ref[1, :N], wm[D + DH:2 * D], preferred_element_type=jnp.float32, precision=lax.Precision.HIGHEST)
    aggr = aggr + jnp.dot(ef, wm[2 * D:], preferred_element_type=jnp.float32, precision=lax.Precision.HIGHEST)
    aggr = aggr + deg * bm_ref[...]
    out = jnp.dot(aggr, wn_ref[...], preferred_element_type=jnp.float32, precision=lax.Precision.HIGHEST) + bn_ref[...]
    mu = jnp.mean(out, axis=0, keepdims=True)
    cen = out - mu
    var = jnp.mean(cen * cen, axis=0, keepdims=True)
    o_ref[...] = cen * lax.rsqrt(var + 1e-5) * gam_ref[...] + bet_ref[...]


_tc_layer = pl.pallas_call(
    _tc_layer_body,
    out_shape=jax.ShapeDtypeStruct((N, D), jnp.float32),
)


def _tc_head_body(h_ref, bat_ref, wl_ref, bl_ref, wg_ref, bg_ref, wf_ref,
                  bf_ref, o_ref):
    h = h_ref[...]
    states = jnp.dot(h, wl_ref[...], preferred_element_type=jnp.float32, precision=lax.Precision.HIGHEST) + bl_ref[...]
    z = jnp.dot(h, wg_ref[...], preferred_element_type=jnp.float32, precision=lax.Precision.HIGHEST) + bg_ref[...]
    z = z - jnp.max(z, axis=1, keepdims=True)
    ez = jnp.exp(z)
    s = states * (ez / jnp.sum(ez, axis=1, keepdims=True))
    onehot = (bat_ref[...] == lax.broadcasted_iota(jnp.int32, (N, B), 1))
    onehot = onehot.astype(jnp.float32)
    sums = lax.dot_general(onehot, s, (((0,), (0,)), ((), ())),
                           preferred_element_type=jnp.float32, precision=lax.Precision.HIGHEST)
    cnt = jnp.sum(onehot, axis=0)[:, None]
    mean = sums / jnp.maximum(cnt, 1.0)
    o_ref[...] = jnp.dot(mean, wf_ref[...], preferred_element_type=jnp.float32, precision=lax.Precision.HIGHEST) + bf_ref[...]


_tc_head = pl.pallas_call(
    _tc_head_body,
    out_shape=jax.ShapeDtypeStruct((B, G), jnp.float32),
)


def _tc_layer3_body(h_ref, sp_ref, wc_ref, c_ref, degb_ref, gam_ref, bet_ref,
                    bat_ref, wl_ref, bl_ref, wg_ref, bg_ref, wf_ref, bf_ref,
                    o_ref, og_ref):
    wc = wc_ref[...]
    out = _dot_hp(degb_ref[...] * h_ref[...], wc[:D])
    out = out + _dot_hp(sp_ref[0, :N], wc[D:D + DH])
    out = out + _dot_hp(sp_ref[1, :N], wc[D + DH:])
    out = out + c_ref[...]
    mu = jnp.mean(out, axis=0, keepdims=True)
    cen = out - mu
    var = jnp.mean(cen * cen, axis=0, keepdims=True)
    h = cen * lax.rsqrt(var + 1e-5) * gam_ref[...] + bet_ref[...]
    o_ref[...] = h
    states = _dot_hp(h, wl_ref[...]) + bl_ref[...]
    z = _dot_hp(h, wg_ref[...]) + bg_ref[...]
    z = z - jnp.max(z, axis=1, keepdims=True)
    ez = jnp.exp(z)
    s = states * (ez / jnp.sum(ez, axis=1, keepdims=True))
    onehot = (bat_ref[...] == lax.broadcasted_iota(jnp.int32, (N, B), 1))
    onehot = onehot.astype(jnp.float32)
    s_hi = s.astype(jnp.bfloat16)
    s_lo = (s - s_hi.astype(jnp.float32)).astype(jnp.bfloat16)
    oh = onehot.astype(jnp.bfloat16)
    dg = lambda x, y: lax.dot_general(x, y, (((0,), (0,)), ((), ())),
                                      preferred_element_type=jnp.float32)
    sums = dg(oh, s_hi) + dg(oh, s_lo)
    cnt = jnp.sum(onehot, axis=0)[:, None]
    mean = sums / jnp.maximum(cnt, 1.0)
    og_ref[...] = _dot_hp(mean, wf_ref[...]) + bf_ref[...]


_tc_layer3 = pl.pallas_call(
    _tc_layer3_body,
    out_shape=(jax.ShapeDtypeStruct((N, D), jnp.float32),
               jax.ShapeDtypeStruct((B, G), jnp.float32)),
)


def kernel(x1, x2, edge_feats, Wm, bm, Wn, bn, gamma, beta, Wl, bl, Wg, bg,
           Wf, bf, edge_index, batch):
    f32 = jnp.float32
    src = edge_index[0].reshape(NS, NCH, CH)
    dst = edge_index[1].reshape(NS, NCH, CH)
    dst_e = edge_index[1].reshape(NW, NCHE, CH)
    ef_r = edge_feats.reshape(NW, NCHE, CH, DE)
    ones_row = jnp.ones((CH, DE), f32)
    zrow_e = jnp.zeros((RPS, DE), f32)

    efp, degp = _sc_ef_scatter(ef_r, dst_e, ones_row, zrow_e)
    Wc, C, degb = _tc_prep(efp, degp, Wm, bm[:, None, :], Wn, bn[:, None, :])
    zrow_d = jnp.minimum(degp[0, :RPS, :1], 0.0) + jnp.zeros((RPS, DH), f32)

    h = x1
    hs = _tc_split(x1)
    for i in range(L):
        sp = _sc_row_scatter(hs[0], hs[1], src, dst, zrow_d)
        if i < L - 1:
            h = _tc_layer(h, sp, Wc[i], C[i], degb, gamma[i][None, :],
                          beta[i][None, :])
            hs = h[:, :DH], h[:, DH:]
        else:
            h, graph = _tc_layer3(h, sp, Wc[i], C[i], degb, gamma[i][None, :],
                                  beta[i][None, :], batch[:, None], Wl,
                                  bl[None, :], Wg, bg[None, :], Wf,
                                  bf[None, :])
    return (h, graph)


# speedup vs baseline: 1.0842x; 1.0842x over previous
import functools

import jax
import jax.numpy as jnp
from jax import lax
from jax.experimental import pallas as pl
from jax.experimental.pallas import tpu as pltpu
from jax.experimental.pallas import tpu_sc as plsc

N = 10000
E = 320000
D = 128
DE = 16
L = 3
G = 128
B = 64

NC, NS = 2, 16
NW = NC * NS
CH = 80
NBUF = 5
NPAD = 10240
RPS = NPAD // NS
DH = D // 2
ESS = E // NS
NCH = ESS // CH
EPW = E // NW
NCHE = EPW // CH

_mesh = plsc.VectorSubcoreMesh(core_axis_name="c", subcore_axis_name="s")


@functools.partial(
    pl.kernel,
    mesh=_mesh,
    compiler_params=pltpu.CompilerParams(use_tc_tiling_on_sc=False),
    out_type=jax.ShapeDtypeStruct((NC, NPAD, DH), jnp.float32),
    scratch_types=(
        [pltpu.VMEM((NCH, CH), jnp.int32),
         pltpu.VMEM((NCH, CH), jnp.int32)]
        + [pltpu.VMEM((CH, DH), jnp.float32) for _ in range(NBUF)]
        + [pltpu.SemaphoreType.DMA for _ in range(2 * NBUF)]
        + [pltpu.VMEM_SHARED((NPAD, DH), jnp.float32)]
    ),
)
def _sc_row_scatter(h0_hbm, h1_hbm, srcr_hbm, dstr_hbm, zrow_hbm, out_hbm,
                    src_v, dst_v, b0, b1, b2, b3, b4, s0, s1, s2, s3, s4,
                    t0, t1, t2, t3, t4, acc_sh):
    bufs = (b0, b1, b2, b3, b4)
    sems = (s0, s1, s2, s3, s4)
    ssems = (t0, t1, t2, t3, t4)
    cid = lax.axis_index("c")
    sid = lax.axis_index("s")

    pltpu.sync_copy(srcr_hbm.at[sid], src_v)
    pltpu.sync_copy(dstr_hbm.at[sid], dst_v)
    pltpu.sync_copy(zrow_hbm, acc_sh.at[pl.ds(sid * RPS, RPS)])
    plsc.subcore_barrier()

    def gather(j, b):
        @pl.when(cid == 0)
        def _():
            pltpu.async_copy(h0_hbm.at[src_v.at[j]], bufs[b], sems[b])

        @pl.when(cid == 1)
        def _():
            pltpu.async_copy(h1_hbm.at[src_v.at[j]], bufs[b], sems[b])

    for b in range(NBUF):
        gather(b, b)

    def body(g, carry):
        for b in range(NBUF):
            j = g * NBUF + b
            pltpu.make_async_copy(h0_hbm.at[src_v.at[j]], bufs[b], sems[b]).wait()
            pltpu.async_copy(bufs[b], acc_sh.at[dst_v.at[j]], ssems[b], add=True)
            nxt = j + NBUF

            @pl.when(nxt < NCH)
            def _():
                pltpu.make_async_copy(bufs[b], acc_sh.at[dst_v.at[0]], ssems[b]).wait()
                gather(nxt, b)
        return carry

    lax.fori_loop(0, NCH // NBUF, body, 0)
    for b in range(NBUF):
        pltpu.make_async_copy(bufs[b], acc_sh.at[dst_v.at[0]], ssems[b]).wait()
    plsc.subcore_barrier()
    pltpu.sync_copy(acc_sh.at[pl.ds(sid * RPS, RPS)],
                    out_hbm.at[cid, pl.ds(sid * RPS, RPS)])


@functools.partial(
    pl.kernel,
    mesh=_mesh,
    compiler_params=pltpu.CompilerParams(use_tc_tiling_on_sc=False),
    out_type=(jax.ShapeDtypeStruct((NC, NPAD, DE), jnp.float32),
              jax.ShapeDtypeStruct((NC, NPAD, DE), jnp.float32)),
    scratch_types=(
        [pltpu.VMEM((NCHE, CH), jnp.int32)]
        + [pltpu.VMEM((CH, DE), jnp.float32) for _ in range(NBUF)]
        + [pltpu.VMEM((CH, DE), jnp.float32)]
        + [pltpu.SemaphoreType.DMA for _ in range(NBUF)]
        + [pltpu.VMEM_SHARED((NPAD, DE), jnp.float32),
           pltpu.VMEM_SHARED((NPAD, DE), jnp.float32)]
    ),
)
def _sc_ef_scatter(ef_hbm, dstr_hbm, ones_hbm, zrow_hbm, oute_hbm, outd_hbm,
                   dst_v, b0, b1, b2, b3, b4, ones_v, s0, s1, s2, s3, s4,
                   acce_sh, accd_sh):
    bufs = (b0, b1, b2, b3, b4)
    sems = (s0, s1, s2, s3, s4)
    cid = lax.axis_index("c")
    sid = lax.axis_index("s")
    wid = sid * NC + cid

    pltpu.sync_copy(dstr_hbm.at[wid], dst_v)
    pltpu.sync_copy(ones_hbm, ones_v)
    pltpu.sync_copy(zrow_hbm, acce_sh.at[pl.ds(sid * RPS, RPS)])
    pltpu.sync_copy(zrow_hbm, accd_sh.at[pl.ds(sid * RPS, RPS)])
    plsc.subcore_barrier()

    for b in range(NBUF):
        pltpu.async_copy(ef_hbm.at[wid, b], bufs[b], sems[b])

    def body(g, carry):
        for b in range(NBUF):
            j = g * NBUF + b
            pltpu.make_async_copy(ef_hbm.at[wid, j], bufs[b], sems[b]).wait()
            pltpu.sync_copy(bufs[b], acce_sh.at[dst_v.at[j]], add=True)
            pltpu.sync_copy(ones_v, accd_sh.at[dst_v.at[j]], add=True)
            nxt = j + NBUF

            @pl.when(nxt < NCHE)
            def _():
                pltpu.async_copy(ef_hbm.at[wid, nxt], bufs[b], sems[b])
        return carry

    lax.fori_loop(0, NCHE // NBUF, body, 0)
    plsc.subcore_barrier()
    pltpu.sync_copy(acce_sh.at[pl.ds(sid * RPS, RPS)],
                    oute_hbm.at[cid, pl.ds(sid * RPS, RPS)])
    pltpu.sync_copy(accd_sh.at[pl.ds(sid * RPS, RPS)],
                    outd_hbm.at[cid, pl.ds(sid * RPS, RPS)])


def _dot_hp(a, b):
    a_hi = a.astype(jnp.bfloat16)
    a_lo = (a - a_hi.astype(jnp.float32)).astype(jnp.bfloat16)
    b_hi = b.astype(jnp.bfloat16)
    b_lo = (b - b_hi.astype(jnp.float32)).astype(jnp.bfloat16)
    d = lambda x, y: jnp.dot(x, y, preferred_element_type=jnp.float32)
    return d(a_hi, b_hi) + (d(a_hi, b_lo) + d(a_lo, b_hi))


def _tc_fold_body(wm_ref, bm_ref, wn_ref, wc_ref, bc_ref):
    for i in range(L):
        wc_ref[i] = _dot_hp(wm_ref[i], wn_ref[i])
        bc_ref[i] = _dot_hp(bm_ref[i], wn_ref[i])


_tc_fold = pl.pallas_call(
    _tc_fold_body,
    out_shape=(jax.ShapeDtypeStruct((L, 2 * D + DE, D), jnp.float32),
               jax.ShapeDtypeStruct((L, 1, D), jnp.float32)),
)


def _tc_layer_body(h_ref, sp_ref, efp_ref, degp_ref, wc_ref, bc_ref, bn_ref,
                   gam_ref, bet_ref, o_ref):
    h = h_ref[...]
    ef = (efp_ref[0] + efp_ref[1])[:N]
    deg = (degp_ref[0, :N, :1] + degp_ref[1, :N, :1])
    wc = wc_ref[...]
    out = _dot_hp(deg * h, wc[:D])
    out = out + _dot_hp(sp_ref[0, :N], wc[D:D + DH])
    out = out + _dot_hp(sp_ref[1, :N], wc[D + DH:2 * D])
    out = out + _dot_hp(ef, wc[2 * D:])
    out = out + deg * bc_ref[...] + bn_ref[...]
    mu = jnp.mean(out, axis=0, keepdims=True)
    cen = out - mu
    var = jnp.mean(cen * cen, axis=0, keepdims=True)
    o_ref[...] = cen * lax.rsqrt(var + 1e-5) * gam_ref[...] + bet_ref[...]


_tc_layer = pl.pallas_call(
    _tc_layer_body,
    out_shape=jax.ShapeDtypeStruct((N, D), jnp.float32),
)


def _tc_head_body(h_ref, bat_ref, wl_ref, bl_ref, wg_ref, bg_ref, wf_ref,
                  bf_ref, o_ref):
    h = h_ref[...]
    states = _dot_hp(h, wl_ref[...]) + bl_ref[...]
    z = _dot_hp(h, wg_ref[...]) + bg_ref[...]
    z = z - jnp.max(z, axis=1, keepdims=True)
    ez = jnp.exp(z)
    s = states * (ez / jnp.sum(ez, axis=1, keepdims=True))
    onehot = (bat_ref[...] == lax.broadcasted_iota(jnp.int32, (N, B), 1))
    onehot = onehot.astype(jnp.float32)
    s_hi = s.astype(jnp.bfloat16)
    s_lo = (s - s_hi.astype(jnp.float32)).astype(jnp.bfloat16)
    oh = onehot.astype(jnp.bfloat16)
    dg = lambda x, y: lax.dot_general(x, y, (((0,), (0,)), ((), ())),
                                      preferred_element_type=jnp.float32)
    sums = dg(oh, s_hi) + dg(oh, s_lo)
    cnt = jnp.sum(onehot, axis=0)[:, None]
    mean = sums / jnp.maximum(cnt, 1.0)
    o_ref[...] = _dot_hp(mean, wf_ref[...]) + bf_ref[...]


_tc_head = pl.pallas_call(
    _tc_head_body,
    out_shape=jax.ShapeDtypeStruct((B, G), jnp.float32),
)


def kernel(x1, x2, edge_feats, Wm, bm, Wn, bn, gamma, beta, Wl, bl, Wg, bg,
           Wf, bf, edge_index, batch):
    f32 = jnp.float32
    src = edge_index[0].reshape(NS, NCH, CH)
    dst = edge_index[1].reshape(NS, NCH, CH)
    dst_e = edge_index[1].reshape(NW, NCHE, CH)
    ef_r = edge_feats.reshape(NW, NCHE, CH, DE)
    ones_row = jnp.ones((CH, DE), f32)
    zrow_d = jnp.zeros((RPS, DH), f32)
    zrow_e = jnp.zeros((RPS, DE), f32)

    efp, degp = _sc_ef_scatter(ef_r, dst_e, ones_row, zrow_e)
    Wc, bc = _tc_fold(Wm, bm[:, None, :], Wn)

    h = x1
    for i in range(L):
        sp = _sc_row_scatter(h[:, :DH], h[:, DH:], src, dst, zrow_d)
        h = _tc_layer(h, sp, efp, degp, Wc[i], bc[i], bn[i][None, :],
                      gamma[i][None, :], beta[i][None, :])

    graph = _tc_head(h, batch[:, None], Wl, bl[None, :], Wg, bg[None, :],
                     Wf, bf[None, :])
    return (h, graph)
